# baseline (speedup 1.0000x reference)
import dataclasses
import functools

import jax
import jax.numpy as jnp
from jax.experimental import pallas as pl
from jax.experimental.pallas import tpu as pltpu
from jax.experimental.pallas import tpu_sc as plsc

_NUM_EXPERTS = 8
_TOP_K = 2
_CAPACITY_FACTOR = 1.0


def _ffn_body(buf_ref, wu_ref, bu_ref, wd_ref, bd_ref, g_ref, o_ref):
    j = pl.program_id(1)
    nj = pl.num_programs(1)
    xb = buf_ref[...].astype(jnp.bfloat16)
    up = jnp.dot(xb, wu_ref[0].astype(jnp.bfloat16),
                 preferred_element_type=jnp.float32)
    up = up + bu_ref[0]
    h = jax.nn.gelu(up)
    yp = jnp.dot(h.astype(jnp.bfloat16), wd_ref[0].astype(jnp.bfloat16),
                 preferred_element_type=jnp.float32)

    @pl.when(j == 0)
    def _():
        o_ref[...] = yp + bd_ref[0]

    @pl.when((j != 0) & (j != nj - 1))
    def _():
        o_ref[...] += yp

    @pl.when((j == nj - 1) & (j != 0))
    def _():
        o_ref[...] = (o_ref[...] + yp) * g_ref[0]


def _expert_ffn(buf, wu, b_up, wd, b_down, gscale, ht):
    ec, d = buf.shape
    e, _, h = wu.shape
    cp = ec // e
    grid = (e, h // ht)
    return pl.pallas_call(
        _ffn_body,
        grid=grid,
        in_specs=[
            pl.BlockSpec((cp, d), lambda i, j: (i, 0)),
            pl.BlockSpec((1, d, ht), lambda i, j: (i, 0, j)),
            pl.BlockSpec((1, 1, ht), lambda i, j: (i, 0, j)),
            pl.BlockSpec((1, ht, d), lambda i, j: (i, j, 0)),
            pl.BlockSpec((1, 1, d), lambda i, j: (i, 0, 0)),
            pl.BlockSpec((1, cp, 1), lambda i, j: (i, 0, 0)),
        ],
        out_specs=pl.BlockSpec((cp, d), lambda i, j: (i, 0)),
        out_shape=jax.ShapeDtypeStruct((ec, d), jnp.float32),
        compiler_params=pltpu.CompilerParams(
            dimension_semantics=("arbitrary", "arbitrary"),
        ),
    )(buf, wu, b_up[:, None, :], wd, b_down[:, None, :],
      gscale.reshape(e, cp, 1))


def _cap_body(cap, cpp, e_ref, p_ref, sid_ref, gv_ref, cnt_ref, carry_ref):
    i = pl.program_id(0)
    ni = pl.num_programs(0)
    nb = e_ref.shape[0]
    ne = carry_ref.shape[1]

    @pl.when(i == 0)
    def _():
        carry_ref[...] = jnp.zeros_like(carry_ref)

    ev = e_ref[...]
    lane = jax.lax.broadcasted_iota(jnp.int32, (nb, ne), 1)
    oh = ev == lane
    row = jax.lax.broadcasted_iota(jnp.int32, (nb, nb), 0)
    col = jax.lax.broadcasted_iota(jnp.int32, (nb, nb), 1)
    tril = (row >= col).astype(jnp.bfloat16)
    cum = jnp.dot(tril, oh.astype(jnp.bfloat16),
                  preferred_element_type=jnp.float32)
    cumi = cum.astype(jnp.int32) + carry_ref[...]
    pos = jnp.sum(jnp.where(oh, cumi, 0), axis=1, keepdims=True) - 1
    keep = pos < cap
    sid_ref[...] = ev * cpp + jnp.where(keep, pos, cap)
    gv_ref[...] = jnp.where(keep, p_ref[...], 0.0)
    carry_ref[...] = cumi[nb - 1:nb, :]

    @pl.when(i == ni - 1)
    def _():
        cnt_ref[...] = jnp.minimum(cumi[nb - 1:nb, :], cap)


def _capacity(e_flat, p_flat, cap, cpp, ne, nb=512):
    n = e_flat.shape[0]
    return pl.pallas_call(
        functools.partial(_cap_body, cap, cpp),
        grid=(n // nb,),
        in_specs=[pl.BlockSpec((nb, 1), lambda i: (i, 0)),
                  pl.BlockSpec((nb, 1), lambda i: (i, 0))],
        out_specs=(pl.BlockSpec((nb, 1), lambda i: (i, 0)),
                   pl.BlockSpec((nb, 1), lambda i: (i, 0)),
                   pl.BlockSpec((1, ne), lambda i: (0, 0))),
        out_shape=(jax.ShapeDtypeStruct((n, 1), jnp.int32),
                   jax.ShapeDtypeStruct((n, 1), jnp.float32),
                   jax.ShapeDtypeStruct((1, ne), jnp.int32)),
        scratch_shapes=[pltpu.VMEM((1, ne), jnp.int32)],
        compiler_params=pltpu.CompilerParams(
            dimension_semantics=("arbitrary",)),
    )(e_flat.reshape(n, 1), p_flat.reshape(n, 1))


def _sc_build_tables(slot_id, tok_flat, gval, ecp):
    n = slot_id.shape[0]
    mesh = plsc.VectorSubcoreMesh(core_axis_name="c", subcore_axis_name="s")
    cp = pltpu.CompilerParams()
    if "needs_layout_passes" in pltpu.CompilerParams.__dataclass_fields__:
        cp = dataclasses.replace(cp, needs_layout_passes=False)

    @functools.partial(
        pl.kernel,
        out_type=(jax.ShapeDtypeStruct((1, ecp), jnp.int32),
                  jax.ShapeDtypeStruct((1, ecp), jnp.float32)),
        mesh=mesh,
        compiler_params=cp,
        scratch_types=[pltpu.VMEM((1, n), jnp.int32),
                       pltpu.VMEM((1, n), jnp.int32),
                       pltpu.VMEM((1, n), jnp.float32),
                       pltpu.VMEM((1, ecp), jnp.int32),
                       pltpu.VMEM((1, ecp), jnp.float32)])
    def bk(sid_hbm, tok_hbm, g_hbm, src_hbm, gs_hbm,
           sid_v, tok_v, g_v, src_v, gs_v):
        @pl.when((jax.lax.axis_index("c") == 0)
                 & (jax.lax.axis_index("s") == 0))
        def _():
            pltpu.sync_copy(sid_hbm, sid_v)
            pltpu.sync_copy(tok_hbm, tok_v)
            pltpu.sync_copy(g_hbm, g_v)

            @pl.loop(0, ecp, step=16)
            def _(i):
                src_v[0, pl.ds(i, 16)] = jnp.zeros((16,), jnp.int32)
                gs_v[0, pl.ds(i, 16)] = jnp.zeros((16,), jnp.float32)

            zero16 = jnp.zeros((16,), jnp.int32)

            @pl.loop(0, n, step=16)
            def _(i):
                idx = sid_v[0, pl.ds(i, 16)]
                plsc.store_scatter(src_v, [zero16, idx],
                                   tok_v[0, pl.ds(i, 16)])
                plsc.store_scatter(gs_v, [zero16, idx],
                                   g_v[0, pl.ds(i, 16)])

            pltpu.sync_copy(src_v, src_hbm)
            pltpu.sync_copy(gs_v, gs_hbm)

    return bk(slot_id.reshape(1, n), tok_flat.reshape(1, n),
              gval.reshape(1, n))


_CW = 16


def _sc_combine(y2d, a0, a1, tok, t, d):
    w = _CW
    mesh = plsc.VectorSubcoreMesh(core_axis_name="c", subcore_axis_name="s")

    nsub = 128 // w

    @functools.partial(
        pl.kernel,
        out_type=jax.ShapeDtypeStruct((t, d), jnp.float32),
        mesh=mesh,
        scratch_types=[pltpu.VMEM((w, d), jnp.float32),
                       pltpu.VMEM((w, d), jnp.float32),
                       pltpu.VMEM((w, d), jnp.float32),
                       pltpu.VMEM((w, d), jnp.float32),
                       pltpu.SemaphoreType.DMA,
                       pltpu.SemaphoreType.DMA,
                       pltpu.SemaphoreType.DMA,
                       pltpu.SemaphoreType.DMA])
    def ck(y_hbm, a0_hbm, a1_hbm, tok_hbm, o_hbm,
           t0a, t1a, t0b, t1b, sm0, sm1, sm2, sm3):
        banks = ((t0a, t1a, sm0, sm1), (t0b, t1b, sm2, sm3))

        def body(a0_v, a1_v, tok_v):
            def start(s, bank):
                tx0, tx1, s0, s1 = bank
                i0 = a0_v[0, pl.ds(s * w, w)]
                i1 = a1_v[0, pl.ds(s * w, w)]
                c0 = pltpu.make_async_copy(y_hbm.at[i0], tx0, s0)
                c1 = pltpu.make_async_copy(y_hbm.at[i1], tx1, s1)
                c0.start()
                c1.start()
                return c0, c1

            inflight = [start(0, banks[0]), None]
            for s in range(nsub):
                cur = s % 2
                if s + 1 < nsub:
                    inflight[(s + 1) % 2] = start(s + 1, banks[(s + 1) % 2])
                c0, c1 = inflight[cur]
                c0.wait()
                c1.wait()
                tx0, tx1 = banks[cur][0], banks[cur][1]

                @pl.loop(0, w)
                def _(r):
                    @pl.loop(0, d, step=16)
                    def _(cc):
                        slc = (pl.ds(r, 1), pl.ds(cc, 16))
                        tx0.at[*slc][...] = (tx0.at[*slc][...]
                                             + tx1.at[*slc][...])

                it = tok_v[0, pl.ds(s * w, w)]
                pltpu.sync_copy(tx0, o_hbm.at[it])

        pltpu.emit_pipeline(
            body,
            grid=(t // 128,),
            in_specs=[pl.BlockSpec((1, 128), lambda i: (0, i)),
                      pl.BlockSpec((1, 128), lambda i: (0, i)),
                      pl.BlockSpec((1, 128), lambda i: (0, i))],
            out_specs=[],
            core_axis_name=("c", "s"),
            dimension_semantics=(pltpu.PARALLEL,),
        )(a0_hbm, a1_hbm, tok_hbm)

    return ck(y2d, a0.reshape(1, t), a1.reshape(1, t), tok.reshape(1, t))


def kernel(x, W_router, W_up, b_up, W_down, b_down):
    B, S, D = x.shape
    T = B * S
    E = _NUM_EXPERTS
    K = _TOP_K
    H = W_up.shape[2]
    C = max(int(_CAPACITY_FACTOR * T / E), K)
    Cp = C + 8

    xr = x.reshape(T, D)
    router_logits = xr @ W_router
    router_z_loss = jnp.mean(jnp.square(
        jax.nn.logsumexp(router_logits, axis=-1, keepdims=True)))
    router_probs = jax.nn.softmax(router_logits, axis=-1)
    top_k_probs, top_k_indices = jax.lax.top_k(router_probs, K)
    top_k_probs = top_k_probs / jnp.sum(top_k_probs, axis=-1, keepdims=True)
    sorted_idx = jnp.argsort(-1.0 * top_k_probs[:, 0])
    e_flat = top_k_indices[sorted_idx].reshape(-1)
    p_flat = top_k_probs[sorted_idx].reshape(-1)
    tok_flat = jnp.repeat(sorted_idx, K)
    sid2, gv2, counts = _capacity(e_flat, p_flat, C, Cp, E)
    slot_id = sid2.reshape(-1)

    frac = counts.reshape(-1).astype(jnp.float32) / float(T * K)
    balance_loss = jnp.mean(jnp.square(frac - 1.0 / E))

    src, gscale = _sc_build_tables(slot_id, tok_flat, gv2.reshape(-1),
                                   E * Cp)
    src = src.reshape(-1)
    gscale = gscale.reshape(-1)

    buf = jnp.take(xr, src, axis=0)

    y2d = _expert_ffn(buf, W_up, b_up, W_down, b_down, gscale, ht=2048)

    ak = slot_id.reshape(T, K)
    out = _sc_combine(y2d, ak[:, 0], ak[:, 1], sorted_idx.astype(jnp.int32),
                      T, D)
    return out.reshape(B, S, D), router_z_loss, balance_loss


# speedup vs baseline: 1.0389x; 1.0389x over previous
import dataclasses
import functools

import jax
import jax.numpy as jnp
from jax.experimental import pallas as pl
from jax.experimental.pallas import tpu as pltpu
from jax.experimental.pallas import tpu_sc as plsc

_NUM_EXPERTS = 8
_TOP_K = 2
_CAPACITY_FACTOR = 1.0


def _ffn_body(buf_ref, wu_ref, bu_ref, wd_ref, bd_ref, g_ref, o_ref):
    j = pl.program_id(1)
    nj = pl.num_programs(1)
    xb = buf_ref[...].astype(jnp.bfloat16)
    up = jnp.dot(xb, wu_ref[0].astype(jnp.bfloat16),
                 preferred_element_type=jnp.float32)
    up = up + bu_ref[0]
    h = jax.nn.gelu(up)
    yp = jnp.dot(h.astype(jnp.bfloat16), wd_ref[0].astype(jnp.bfloat16),
                 preferred_element_type=jnp.float32)

    @pl.when(j == 0)
    def _():
        o_ref[...] = yp + bd_ref[0]

    @pl.when((j != 0) & (j != nj - 1))
    def _():
        o_ref[...] += yp

    @pl.when((j == nj - 1) & (j != 0))
    def _():
        o_ref[...] = (o_ref[...] + yp) * g_ref[0]


def _expert_ffn(buf, wu, b_up, wd, b_down, gscale, ht):
    ec, d = buf.shape
    e, _, h = wu.shape
    cp = ec // e
    grid = (e, h // ht)
    return pl.pallas_call(
        _ffn_body,
        grid=grid,
        in_specs=[
            pl.BlockSpec((cp, d), lambda i, j: (i, 0)),
            pl.BlockSpec((1, d, ht), lambda i, j: (i, 0, j)),
            pl.BlockSpec((1, 1, ht), lambda i, j: (i, 0, j)),
            pl.BlockSpec((1, ht, d), lambda i, j: (i, j, 0)),
            pl.BlockSpec((1, 1, d), lambda i, j: (i, 0, 0)),
            pl.BlockSpec((1, cp, 1), lambda i, j: (i, 0, 0)),
        ],
        out_specs=pl.BlockSpec((cp, d), lambda i, j: (i, 0)),
        out_shape=jax.ShapeDtypeStruct((ec, d), jnp.float32),
        compiler_params=pltpu.CompilerParams(
            dimension_semantics=("arbitrary", "arbitrary"),
        ),
    )(buf, wu, b_up[:, None, :], wd, b_down[:, None, :],
      gscale.reshape(e, cp, 1))


def _sc_build_tables(slot_id, tok_flat, gval, ecp):
    n = slot_id.shape[0]
    mesh = plsc.VectorSubcoreMesh(core_axis_name="c", subcore_axis_name="s")
    cp = pltpu.CompilerParams()
    if "needs_layout_passes" in pltpu.CompilerParams.__dataclass_fields__:
        cp = dataclasses.replace(cp, needs_layout_passes=False)

    @functools.partial(
        pl.kernel,
        out_type=(jax.ShapeDtypeStruct((1, ecp), jnp.int32),
                  jax.ShapeDtypeStruct((1, ecp), jnp.float32)),
        mesh=mesh,
        compiler_params=cp,
        scratch_types=[pltpu.VMEM((1, n), jnp.int32),
                       pltpu.VMEM((1, n), jnp.int32),
                       pltpu.VMEM((1, n), jnp.float32),
                       pltpu.VMEM((1, ecp), jnp.int32),
                       pltpu.VMEM((1, ecp), jnp.float32)])
    def bk(sid_hbm, tok_hbm, g_hbm, src_hbm, gs_hbm,
           sid_v, tok_v, g_v, src_v, gs_v):
        @pl.when((jax.lax.axis_index("c") == 0)
                 & (jax.lax.axis_index("s") == 0))
        def _():
            pltpu.sync_copy(sid_hbm, sid_v)
            pltpu.sync_copy(tok_hbm, tok_v)
            pltpu.sync_copy(g_hbm, g_v)

            @pl.loop(0, ecp, step=16)
            def _(i):
                src_v[0, pl.ds(i, 16)] = jnp.zeros((16,), jnp.int32)
                gs_v[0, pl.ds(i, 16)] = jnp.zeros((16,), jnp.float32)

            zero16 = jnp.zeros((16,), jnp.int32)

            @pl.loop(0, n, step=16)
            def _(i):
                idx = sid_v[0, pl.ds(i, 16)]
                plsc.store_scatter(src_v, [zero16, idx],
                                   tok_v[0, pl.ds(i, 16)])
                plsc.store_scatter(gs_v, [zero16, idx],
                                   g_v[0, pl.ds(i, 16)])

            pltpu.sync_copy(src_v, src_hbm)
            pltpu.sync_copy(gs_v, gs_hbm)

    return bk(slot_id.reshape(1, n), tok_flat.reshape(1, n),
              gval.reshape(1, n))


_CW = 16


def _sc_combine(y2d, a0, a1, tok, t, d):
    w = _CW
    mesh = plsc.VectorSubcoreMesh(core_axis_name="c", subcore_axis_name="s")

    nsub = 128 // w

    @functools.partial(
        pl.kernel,
        out_type=jax.ShapeDtypeStruct((t, d), jnp.float32),
        mesh=mesh,
        scratch_types=[pltpu.VMEM((w, d), jnp.float32),
                       pltpu.VMEM((w, d), jnp.float32),
                       pltpu.VMEM((w, d), jnp.float32),
                       pltpu.VMEM((w, d), jnp.float32),
                       pltpu.SemaphoreType.DMA,
                       pltpu.SemaphoreType.DMA,
                       pltpu.SemaphoreType.DMA,
                       pltpu.SemaphoreType.DMA])
    def ck(y_hbm, a0_hbm, a1_hbm, tok_hbm, o_hbm,
           t0a, t1a, t0b, t1b, sm0, sm1, sm2, sm3):
        banks = ((t0a, t1a, sm0, sm1), (t0b, t1b, sm2, sm3))

        def body(a0_v, a1_v, tok_v):
            def start(s, bank):
                tx0, tx1, s0, s1 = bank
                i0 = a0_v[0, pl.ds(s * w, w)]
                i1 = a1_v[0, pl.ds(s * w, w)]
                c0 = pltpu.make_async_copy(y_hbm.at[i0], tx0, s0)
                c1 = pltpu.make_async_copy(y_hbm.at[i1], tx1, s1)
                c0.start()
                c1.start()
                return c0, c1

            inflight = [start(0, banks[0]), None]
            for s in range(nsub):
                cur = s % 2
                if s + 1 < nsub:
                    inflight[(s + 1) % 2] = start(s + 1, banks[(s + 1) % 2])
                c0, c1 = inflight[cur]
                c0.wait()
                c1.wait()
                tx0, tx1 = banks[cur][0], banks[cur][1]

                @pl.loop(0, w)
                def _(r):
                    @pl.loop(0, d, step=16)
                    def _(cc):
                        slc = (pl.ds(r, 1), pl.ds(cc, 16))
                        tx0.at[*slc][...] = (tx0.at[*slc][...]
                                             + tx1.at[*slc][...])

                it = tok_v[0, pl.ds(s * w, w)]
                pltpu.sync_copy(tx0, o_hbm.at[it])

        pltpu.emit_pipeline(
            body,
            grid=(t // 128,),
            in_specs=[pl.BlockSpec((1, 128), lambda i: (0, i)),
                      pl.BlockSpec((1, 128), lambda i: (0, i)),
                      pl.BlockSpec((1, 128), lambda i: (0, i))],
            out_specs=[],
            core_axis_name=("c", "s"),
            dimension_semantics=(pltpu.PARALLEL,),
        )(a0_hbm, a1_hbm, tok_hbm)

    return ck(y2d, a0.reshape(1, t), a1.reshape(1, t), tok.reshape(1, t))


def kernel(x, W_router, W_up, b_up, W_down, b_down):
    B, S, D = x.shape
    T = B * S
    E = _NUM_EXPERTS
    K = _TOP_K
    H = W_up.shape[2]
    C = max(int(_CAPACITY_FACTOR * T / E), K)
    Cp = C + 8

    xr = x.reshape(T, D)
    router_logits = xr @ W_router
    router_z_loss = jnp.mean(jnp.square(
        jax.nn.logsumexp(router_logits, axis=-1, keepdims=True)))
    router_probs = jax.nn.softmax(router_logits, axis=-1)
    top_k_probs, top_k_indices = jax.lax.top_k(router_probs, K)
    top_k_probs = top_k_probs / jnp.sum(top_k_probs, axis=-1, keepdims=True)
    sorted_idx = jnp.argsort(-1.0 * top_k_probs[:, 0])
    e_flat = top_k_indices[sorted_idx].reshape(-1)
    p_flat = top_k_probs[sorted_idx].reshape(-1)
    tok_flat = jnp.repeat(sorted_idx, K)
    one_hot = jax.nn.one_hot(e_flat, E, dtype=jnp.int32)
    pos = jnp.sum(jnp.cumsum(one_hot, axis=0) * one_hot, axis=1) - 1
    keep = pos < C
    slot = jnp.where(keep, pos, C)

    counts = jnp.sum(one_hot * keep[:, None].astype(jnp.int32), axis=0)
    frac = counts.astype(jnp.float32) / float(T * K)
    balance_loss = jnp.mean(jnp.square(frac - 1.0 / E))

    slot_id = e_flat * Cp + slot
    src, gscale = _sc_build_tables(slot_id, tok_flat,
                                   jnp.where(keep, p_flat, 0.0), E * Cp)
    src = src.reshape(-1)
    gscale = gscale.reshape(-1)

    buf = jnp.take(xr, src, axis=0)

    y2d = _expert_ffn(buf, W_up, b_up, W_down, b_down, gscale, ht=2048)

    ak = slot_id.reshape(T, K)
    out = _sc_combine(y2d, ak[:, 0], ak[:, 1], sorted_idx.astype(jnp.int32),
                      T, D)
    return out.reshape(B, S, D), router_z_loss, balance_loss
